# baseline (speedup 1.0000x reference)
import functools

import jax
import jax.numpy as jnp
from jax import lax
from jax.experimental import pallas as pl
from jax.experimental.pallas import tpu as pltpu
from jax.experimental.pallas import tpu_sc as plsc

HIDDEN = 128
NLANE = 16
LPAD = 56


def _pos_encoding(length, hidden_size):
    pos = jnp.arange(length, dtype=jnp.float32)[:, None]
    i = jnp.arange(hidden_size // 2, dtype=jnp.float32)[None, :]
    angle_rates = 1.0 / jnp.power(10000.0, (2.0 * i) / jnp.float32(hidden_size))
    angles = pos * angle_rates
    return jnp.concatenate([jnp.sin(angles), jnp.cos(angles)], axis=-1)


def _sc_gather_call(b, l, nw, q_nchunk, q_chunk, v_nchunk, v_chunk):
    v_rows_per_w = b * l // nw
    mesh = plsc.VectorSubcoreMesh(core_axis_name="c", subcore_axis_name="s")

    q_nbuf = 2
    v_nbuf = 5
    assert (q_nchunk - q_nbuf) % q_nbuf == 0
    assert (v_nchunk - v_nbuf) % v_nbuf == 0

    @functools.partial(
        pl.kernel,
        mesh=mesh,
        out_type=[
            jax.ShapeDtypeStruct((b * LPAD // q_chunk, q_chunk, HIDDEN),
                                 jnp.float32),
            jax.ShapeDtypeStruct((b * l // v_chunk, v_chunk, HIDDEN),
                                 jnp.float32),
        ],
        scratch_types=[
            pltpu.VMEM((q_nchunk, q_chunk), jnp.int32),
            pltpu.VMEM((v_nchunk, v_chunk), jnp.int32),
            [pltpu.VMEM((q_chunk, HIDDEN), jnp.float32)
             for _ in range(q_nbuf)],
            [pltpu.VMEM((v_chunk, HIDDEN), jnp.float32)
             for _ in range(v_nbuf)],
            pltpu.VMEM((l, HIDDEN), jnp.float32),
            [pltpu.SemaphoreType.DMA for _ in range(q_nbuf)],
            [pltpu.SemaphoreType.DMA for _ in range(q_nbuf)],
            [pltpu.SemaphoreType.DMA for _ in range(v_nbuf)],
            [pltpu.SemaphoreType.DMA for _ in range(v_nbuf)],
        ],
    )
    def k(qidx_hbm, vidx_hbm, tgt_hbm, src_hbm, pos_hbm, qout_hbm, vout_hbm,
          qidx_v, vidx_v, qbufs, vbufs, pos_v, qgs, qws, vgs, vws):
        wid = lax.axis_index("s") * 2 + lax.axis_index("c")
        pltpu.sync_copy(qidx_hbm.at[wid], qidx_v)
        pltpu.sync_copy(vidx_hbm.at[wid], vidx_v)
        pltpu.sync_copy(pos_hbm, pos_v)

        def q_gather(c, bf):
            pltpu.make_async_copy(tgt_hbm.at[qidx_v.at[c]], qbufs[bf],
                                  qgs[bf]).start()

        def v_gather(c, bf):
            pltpu.make_async_copy(src_hbm.at[vidx_v.at[c]], vbufs[bf],
                                  vgs[bf]).start()

        def q_emit(c, bf):
            pltpu.make_async_copy(tgt_hbm.at[qidx_v.at[c]], qbufs[bf],
                                  qgs[bf]).wait()
            wcp = pltpu.make_async_copy(qbufs[bf],
                                        qout_hbm.at[wid * q_nchunk + c],
                                        qws[bf])
            wcp.start()
            return wcp

        def v_emit(c, bf):
            g = wid * v_rows_per_w + c * v_chunk
            l_idx = g // b
            pltpu.make_async_copy(src_hbm.at[vidx_v.at[c]], vbufs[bf],
                                  vgs[bf]).wait()
            prows = [pos_v[l_idx, pl.ds(h * NLANE, NLANE)]
                     for h in range(HIDDEN // NLANE)]
            vbuf = vbufs[bf]

            def addrow(r, inner, vbuf=vbuf, prows=prows):
                for h in range(HIDDEN // NLANE):
                    sl = pl.ds(h * NLANE, NLANE)
                    vbuf[r, sl] = vbuf[r, sl] + prows[h]
                return inner

            lax.fori_loop(0, v_chunk, addrow, 0)
            wcp = pltpu.make_async_copy(vbuf,
                                        vout_hbm.at[wid * v_nchunk + c],
                                        vws[bf])
            wcp.start()
            return wcp

        for bf in range(q_nbuf):
            q_gather(bf, bf)
        for bf in range(v_nbuf):
            v_gather(bf, bf)

        def q_super(cc, carry):
            for bf in range(q_nbuf):
                c = cc * q_nbuf + bf
                wcp = q_emit(c, bf)
                wcp.wait()
                q_gather(c + q_nbuf, bf)
            return carry

        lax.fori_loop(0, (q_nchunk - q_nbuf) // q_nbuf, q_super, 0)
        for bf in range(q_nbuf):
            q_emit(q_nchunk - q_nbuf + bf, bf).wait()

        def v_super(cc, carry):
            for bf in range(v_nbuf):
                c = cc * v_nbuf + bf
                wcp = v_emit(c, bf)
                wcp.wait()
                v_gather(c + v_nbuf, bf)
            return carry

        lax.fori_loop(0, (v_nchunk - v_nbuf) // v_nbuf, v_super, 0)
        for bf in range(v_nbuf):
            v_emit(v_nchunk - v_nbuf + bf, bf).wait()

    return k


def _tc_matmul_call(b, l, nb):
    def body(ap_ref, qr_ref, pos_ref, out_ref):
        ap = ap_ref[...]
        qr = qr_ref[:, :l, :]
        acc = lax.dot_general(
            ap, qr, (((2,), (1,)), ((0,), (0,))),
            preferred_element_type=jnp.float32)
        out_ref[...] = jnp.transpose(acc, (1, 0, 2)) + pos_ref[...]

    return pl.pallas_call(
        body,
        grid=(b // nb,),
        in_specs=[
            pl.BlockSpec((nb, l, l), lambda i: (i, 0, 0)),
            pl.BlockSpec((nb, LPAD, HIDDEN), lambda i: (i, 0, 0)),
            pl.BlockSpec((l, 1, HIDDEN), lambda i: (0, 0, 0)),
        ],
        out_specs=pl.BlockSpec((l, nb, HIDDEN), lambda i: (0, i, 0)),
        out_shape=jax.ShapeDtypeStruct((l, b, HIDDEN), jnp.float32),
    )


def kernel(queries, values, absolute_positions, src_table, tgt_table):
    b, l = queries.shape
    nw = 32

    pad_cols = (jnp.arange(LPAD - l, dtype=jnp.int32)[None, :]
                + (LPAD - l) * jnp.arange(b, dtype=jnp.int32)[:, None])
    q_idx_pad = jnp.concatenate(
        [queries.astype(jnp.int32), pad_cols], axis=1)
    q_chunk = 2 * LPAD
    q_nchunk = b // (2 * nw)
    q_idx = q_idx_pad.reshape(nw, q_nchunk, q_chunk)

    v_chunk = 64
    v_nchunk = (b * l) // (nw * v_chunk)
    v_idx = values.astype(jnp.int32).T.reshape(nw, v_nchunk, v_chunk)

    pos = _pos_encoding(l, HIDDEN)

    sc = _sc_gather_call(b, l, nw, q_nchunk, q_chunk, v_nchunk, v_chunk)
    q_rows, v_emb_t = sc(q_idx, v_idx, tgt_table, src_table, pos)
    q_rows = q_rows.reshape(b * LPAD, HIDDEN)
    v_emb_t = v_emb_t.reshape(b * l, HIDDEN)

    tc = _tc_matmul_call(b, l, nb=8)
    q_emb_t = tc(absolute_positions, q_rows.reshape(b, LPAD, HIDDEN),
                 pos[:, None, :])

    q_emb = q_emb_t.transpose(1, 0, 2)
    v_emb = v_emb_t.reshape(l, b, HIDDEN).transpose(1, 0, 2)
    return q_emb, v_emb


# speedup vs baseline: 1.2543x; 1.2543x over previous
import functools

import jax
import jax.numpy as jnp
from jax import lax
from jax.experimental import pallas as pl
from jax.experimental.pallas import tpu as pltpu
from jax.experimental.pallas import tpu_sc as plsc

HIDDEN = 128
NLANE = 16
LPAD = 56


def _pos_encoding(length, hidden_size):
    pos = jnp.arange(length, dtype=jnp.float32)[:, None]
    i = jnp.arange(hidden_size // 2, dtype=jnp.float32)[None, :]
    angle_rates = 1.0 / jnp.power(10000.0, (2.0 * i) / jnp.float32(hidden_size))
    angles = pos * angle_rates
    return jnp.concatenate([jnp.sin(angles), jnp.cos(angles)], axis=-1)


def _sc_gather_call(b, l, nw, q_nchunk, q_chunk, v_nchunk, v_chunk):
    v_rows_per_w = b * l // nw
    mesh = plsc.VectorSubcoreMesh(core_axis_name="c", subcore_axis_name="s")

    q_nbuf = 2
    v_nbuf = 5
    assert (q_nchunk - q_nbuf) % q_nbuf == 0
    assert (v_nchunk - v_nbuf) % v_nbuf == 0

    @functools.partial(
        pl.kernel,
        mesh=mesh,
        out_type=[
            jax.ShapeDtypeStruct((b * LPAD // q_chunk, q_chunk, HIDDEN),
                                 jnp.float32),
            jax.ShapeDtypeStruct((b * l // v_chunk, v_chunk, HIDDEN),
                                 jnp.float32),
        ],
        scratch_types=[
            pltpu.VMEM((q_nchunk, q_chunk), jnp.int32),
            pltpu.VMEM((v_nchunk, v_chunk), jnp.int32),
            [pltpu.VMEM((q_chunk, HIDDEN), jnp.float32)
             for _ in range(q_nbuf)],
            [pltpu.VMEM((v_chunk, HIDDEN), jnp.float32)
             for _ in range(v_nbuf)],
            pltpu.VMEM((l, HIDDEN), jnp.float32),
            [pltpu.SemaphoreType.DMA for _ in range(q_nbuf)],
            [pltpu.SemaphoreType.DMA for _ in range(q_nbuf)],
            [pltpu.SemaphoreType.DMA for _ in range(v_nbuf)],
            [pltpu.SemaphoreType.DMA for _ in range(v_nbuf)],
        ],
    )
    def k(qidx_hbm, vidx_hbm, tgt_hbm, src_hbm, pos_hbm, qout_hbm, vout_hbm,
          qidx_v, vidx_v, qbufs, vbufs, pos_v, qgs, qws, vgs, vws):
        wid = lax.axis_index("s") * 2 + lax.axis_index("c")
        pltpu.sync_copy(qidx_hbm.at[wid], qidx_v)
        pltpu.sync_copy(vidx_hbm.at[wid], vidx_v)
        pltpu.sync_copy(pos_hbm, pos_v)

        def q_gather(c, bf):
            pltpu.make_async_copy(tgt_hbm.at[qidx_v.at[c]], qbufs[bf],
                                  qgs[bf]).start()

        def v_gather(c, bf):
            pltpu.make_async_copy(src_hbm.at[vidx_v.at[c]], vbufs[bf],
                                  vgs[bf]).start()

        def q_emit(c, bf):
            pltpu.make_async_copy(tgt_hbm.at[qidx_v.at[c]], qbufs[bf],
                                  qgs[bf]).wait()
            wcp = pltpu.make_async_copy(qbufs[bf],
                                        qout_hbm.at[wid * q_nchunk + c],
                                        qws[bf])
            wcp.start()
            return wcp

        def v_emit(c, bf):
            g = wid * v_rows_per_w + c * v_chunk
            l_idx = g // b
            pltpu.make_async_copy(src_hbm.at[vidx_v.at[c]], vbufs[bf],
                                  vgs[bf]).wait()
            prows = [pos_v[l_idx, pl.ds(h * NLANE, NLANE)]
                     for h in range(HIDDEN // NLANE)]
            vbuf = vbufs[bf]

            def addrow(r, inner, vbuf=vbuf, prows=prows):
                for h in range(HIDDEN // NLANE):
                    sl = pl.ds(h * NLANE, NLANE)
                    vbuf[r, sl] = vbuf[r, sl] + prows[h]
                return inner

            lax.fori_loop(0, v_chunk, addrow, 0)
            wcp = pltpu.make_async_copy(vbuf,
                                        vout_hbm.at[wid * v_nchunk + c],
                                        vws[bf])
            wcp.start()
            return wcp

        for bf in range(q_nbuf):
            q_gather(bf, bf)
        for bf in range(v_nbuf):
            v_gather(bf, bf)

        def q_super(cc, carry):
            for bf in range(q_nbuf):
                c = cc * q_nbuf + bf
                wcp = q_emit(c, bf)
                wcp.wait()
                q_gather(c + q_nbuf, bf)
            return carry

        lax.fori_loop(0, (q_nchunk - q_nbuf) // q_nbuf, q_super, 0)
        for bf in range(q_nbuf):
            q_emit(q_nchunk - q_nbuf + bf, bf).wait()

        def v_super(cc, carry):
            for bf in range(v_nbuf):
                c = cc * v_nbuf + bf
                wcp = v_emit(c, bf)
                wcp.wait()
                v_gather(c + v_nbuf, bf)
            return carry

        lax.fori_loop(0, (v_nchunk - v_nbuf) // v_nbuf, v_super, 0)
        for bf in range(v_nbuf):
            v_emit(v_nchunk - v_nbuf + bf, bf).wait()

    return k


def _tc_matmul_call(b, l, nb):
    def body(ap_ref, qr_ref, pos_ref, out_ref):
        ap = ap_ref[...]
        qr = qr_ref[:, :l, :]
        acc = lax.dot_general(
            ap, qr, (((2,), (1,)), ((0,), (0,))),
            preferred_element_type=jnp.float32)
        out_ref[...] = jnp.transpose(acc, (1, 0, 2)) + pos_ref[...]

    return pl.pallas_call(
        body,
        grid=(b // nb,),
        in_specs=[
            pl.BlockSpec((nb, l, l), lambda i: (i, 0, 0)),
            pl.BlockSpec((nb, LPAD, HIDDEN), lambda i: (i, 0, 0)),
            pl.BlockSpec((l, 1, HIDDEN), lambda i: (0, 0, 0)),
        ],
        out_specs=pl.BlockSpec((l, nb, HIDDEN), lambda i: (0, i, 0)),
        out_shape=jax.ShapeDtypeStruct((l, b, HIDDEN), jnp.float32),
    )


def kernel(queries, values, absolute_positions, src_table, tgt_table):
    b, l = queries.shape
    nw = 32

    pad_cols = (jnp.arange(LPAD - l, dtype=jnp.int32)[None, :]
                + (LPAD - l) * jnp.arange(b, dtype=jnp.int32)[:, None])
    q_idx_pad = jnp.concatenate(
        [queries.astype(jnp.int32), pad_cols], axis=1)
    q_chunk = 2 * LPAD
    q_nchunk = b // (2 * nw)
    q_idx = q_idx_pad.reshape(nw, q_nchunk, q_chunk)

    v_chunk = 64
    v_nchunk = (b * l) // (nw * v_chunk)
    v_idx = values.astype(jnp.int32).T.reshape(nw, v_nchunk, v_chunk)

    pos = _pos_encoding(l, HIDDEN)

    sc = _sc_gather_call(b, l, nw, q_nchunk, q_chunk, v_nchunk, v_chunk)
    q_rows, v_emb_t = sc(q_idx, v_idx, tgt_table, src_table, pos)
    q_rows = q_rows.reshape(b * LPAD, HIDDEN)
    v_emb_t = v_emb_t.reshape(b * l, HIDDEN)

    tc = _tc_matmul_call(b, l, nb=16)
    q_emb_t = tc(absolute_positions, q_rows.reshape(b, LPAD, HIDDEN),
                 pos[:, None, :])

    q_emb = q_emb_t.transpose(1, 0, 2)
    v_emb = v_emb_t.reshape(l, b, HIDDEN).transpose(1, 0, 2)
    return q_emb, v_emb


# speedup vs baseline: 1.4294x; 1.1396x over previous
import functools

import jax
import jax.numpy as jnp
from jax import lax
from jax.experimental import pallas as pl
from jax.experimental.pallas import tpu as pltpu
from jax.experimental.pallas import tpu_sc as plsc

HIDDEN = 128
NLANE = 16
LPAD = 56


def _pos_encoding(length, hidden_size):
    pos = jnp.arange(length, dtype=jnp.float32)[:, None]
    i = jnp.arange(hidden_size // 2, dtype=jnp.float32)[None, :]
    angle_rates = 1.0 / jnp.power(10000.0, (2.0 * i) / jnp.float32(hidden_size))
    angles = pos * angle_rates
    return jnp.concatenate([jnp.sin(angles), jnp.cos(angles)], axis=-1)


def _sc_gather_call(b, l, nw, q_nchunk, q_chunk, v_nchunk, v_chunk):
    v_rows_per_w = b * l // nw
    mesh = plsc.VectorSubcoreMesh(core_axis_name="c", subcore_axis_name="s")

    q_nbuf = 2
    v_nbuf = 5
    assert (q_nchunk - q_nbuf) % q_nbuf == 0
    assert (v_nchunk - v_nbuf) % v_nbuf == 0

    @functools.partial(
        pl.kernel,
        mesh=mesh,
        out_type=[
            jax.ShapeDtypeStruct((b * LPAD // q_chunk, q_chunk, HIDDEN),
                                 jnp.float32),
            jax.ShapeDtypeStruct((b * l // v_chunk, v_chunk, HIDDEN),
                                 jnp.float32),
        ],
        scratch_types=[
            pltpu.VMEM((q_nchunk, q_chunk), jnp.int32),
            pltpu.VMEM((v_nchunk, v_chunk), jnp.int32),
            [pltpu.VMEM((q_chunk, HIDDEN), jnp.float32)
             for _ in range(q_nbuf)],
            [pltpu.VMEM((v_chunk, HIDDEN), jnp.float32)
             for _ in range(v_nbuf)],
            pltpu.VMEM((l, HIDDEN), jnp.float32),
            [pltpu.SemaphoreType.DMA for _ in range(q_nbuf)],
            [pltpu.SemaphoreType.DMA for _ in range(q_nbuf)],
            [pltpu.SemaphoreType.DMA for _ in range(v_nbuf)],
            [pltpu.SemaphoreType.DMA for _ in range(v_nbuf)],
        ],
    )
    def k(qidx_hbm, vidx_hbm, tgt_hbm, src_hbm, pos_hbm, qout_hbm, vout_hbm,
          qidx_v, vidx_v, qbufs, vbufs, pos_v, qgs, qws, vgs, vws):
        wid = lax.axis_index("s") * 2 + lax.axis_index("c")
        pltpu.sync_copy(qidx_hbm.at[wid], qidx_v)
        pltpu.sync_copy(vidx_hbm.at[wid], vidx_v)
        pltpu.sync_copy(pos_hbm, pos_v)

        def q_gather(c, bf):
            pltpu.make_async_copy(tgt_hbm.at[qidx_v.at[c]], qbufs[bf],
                                  qgs[bf]).start()

        def v_gather(c, bf):
            pltpu.make_async_copy(src_hbm.at[vidx_v.at[c]], vbufs[bf],
                                  vgs[bf]).start()

        def q_emit(c, bf):
            pltpu.make_async_copy(tgt_hbm.at[qidx_v.at[c]], qbufs[bf],
                                  qgs[bf]).wait()
            wcp = pltpu.make_async_copy(qbufs[bf],
                                        qout_hbm.at[wid * q_nchunk + c],
                                        qws[bf])
            wcp.start()
            return wcp

        def v_emit(c, bf):
            g = wid * v_rows_per_w + c * v_chunk
            l_idx = g // b
            pltpu.make_async_copy(src_hbm.at[vidx_v.at[c]], vbufs[bf],
                                  vgs[bf]).wait()
            prows = [pos_v[l_idx, pl.ds(h * NLANE, NLANE)]
                     for h in range(HIDDEN // NLANE)]
            vbuf = vbufs[bf]

            def addrow(r, inner, vbuf=vbuf, prows=prows):
                for h in range(HIDDEN // NLANE):
                    sl = pl.ds(h * NLANE, NLANE)
                    vbuf[r, sl] = vbuf[r, sl] + prows[h]
                return inner

            lax.fori_loop(0, v_chunk, addrow, 0)
            wcp = pltpu.make_async_copy(vbuf,
                                        vout_hbm.at[wid * v_nchunk + c],
                                        vws[bf])
            wcp.start()
            return wcp

        for bf in range(q_nbuf):
            q_gather(bf, bf)
        for bf in range(v_nbuf):
            v_gather(bf, bf)

        def q_super(cc, carry):
            for bf in range(q_nbuf):
                c = cc * q_nbuf + bf
                wcp = q_emit(c, bf)
                wcp.wait()
                q_gather(c + q_nbuf, bf)
            return carry

        lax.fori_loop(0, (q_nchunk - q_nbuf) // q_nbuf, q_super, 0)
        for bf in range(q_nbuf):
            q_emit(q_nchunk - q_nbuf + bf, bf).wait()

        def v_super(cc, carry):
            for bf in range(v_nbuf):
                c = cc * v_nbuf + bf
                wcp = v_emit(c, bf)
                wcp.wait()
                v_gather(c + v_nbuf, bf)
            return carry

        lax.fori_loop(0, (v_nchunk - v_nbuf) // v_nbuf, v_super, 0)
        for bf in range(v_nbuf):
            v_emit(v_nchunk - v_nbuf + bf, bf).wait()

    return k


def _tc_matmul_call(b, l, nb):
    def body(ap_ref, qr_ref, pos_ref, out_ref):
        ap = ap_ref[...]
        qr = qr_ref[:, :l, :]
        acc = lax.dot_general(
            ap, qr, (((2,), (1,)), ((0,), (0,))),
            preferred_element_type=jnp.float32)
        out_ref[...] = jnp.transpose(acc, (1, 0, 2)) + pos_ref[...]

    return pl.pallas_call(
        body,
        grid=(b // nb,),
        in_specs=[
            pl.BlockSpec((nb, l, l), lambda i: (i, 0, 0)),
            pl.BlockSpec((nb, LPAD, HIDDEN), lambda i: (i, 0, 0)),
            pl.BlockSpec((l, 1, HIDDEN), lambda i: (0, 0, 0)),
        ],
        out_specs=pl.BlockSpec((l, nb, HIDDEN), lambda i: (0, i, 0)),
        out_shape=jax.ShapeDtypeStruct((l, b, HIDDEN), jnp.float32),
    )


def kernel(queries, values, absolute_positions, src_table, tgt_table):
    b, l = queries.shape
    nw = 32

    pad_cols = (jnp.arange(LPAD - l, dtype=jnp.int32)[None, :]
                + (LPAD - l) * jnp.arange(b, dtype=jnp.int32)[:, None])
    q_idx_pad = jnp.concatenate(
        [queries.astype(jnp.int32), pad_cols], axis=1)
    q_chunk = 2 * LPAD
    q_nchunk = b // (2 * nw)
    q_idx = q_idx_pad.reshape(nw, q_nchunk, q_chunk)

    v_chunk = 64
    v_nchunk = (b * l) // (nw * v_chunk)
    v_idx = values.astype(jnp.int32).T.reshape(nw, v_nchunk, v_chunk)

    pos = _pos_encoding(l, HIDDEN)

    sc = _sc_gather_call(b, l, nw, q_nchunk, q_chunk, v_nchunk, v_chunk)
    q_rows, v_emb_t = sc(q_idx, v_idx, tgt_table, src_table, pos)
    q_rows = q_rows.reshape(b * LPAD, HIDDEN)
    v_emb_t = v_emb_t.reshape(b * l, HIDDEN)

    tc = _tc_matmul_call(b, l, nb=32)
    q_emb_t = tc(absolute_positions, q_rows.reshape(b, LPAD, HIDDEN),
                 pos[:, None, :])

    q_emb = q_emb_t.transpose(1, 0, 2)
    v_emb = v_emb_t.reshape(l, b, HIDDEN).transpose(1, 0, 2)
    return q_emb, v_emb


# speedup vs baseline: 1.5521x; 1.0858x over previous
import functools

import jax
import jax.numpy as jnp
from jax import lax
from jax.experimental import pallas as pl
from jax.experimental.pallas import tpu as pltpu
from jax.experimental.pallas import tpu_sc as plsc

HIDDEN = 128
NLANE = 16
LPAD = 56


def _pos_encoding(length, hidden_size):
    pos = jnp.arange(length, dtype=jnp.float32)[:, None]
    i = jnp.arange(hidden_size // 2, dtype=jnp.float32)[None, :]
    angle_rates = 1.0 / jnp.power(10000.0, (2.0 * i) / jnp.float32(hidden_size))
    angles = pos * angle_rates
    return jnp.concatenate([jnp.sin(angles), jnp.cos(angles)], axis=-1)


def _sc_gather_call(b, l, nw, q_nchunk, q_chunk, v_nchunk, v_chunk):
    v_rows_per_w = b * l // nw
    mesh = plsc.VectorSubcoreMesh(core_axis_name="c", subcore_axis_name="s")

    q_nbuf = 2
    v_nbuf = 5
    assert (q_nchunk - q_nbuf) % q_nbuf == 0
    assert (v_nchunk - v_nbuf) % v_nbuf == 0

    @functools.partial(
        pl.kernel,
        mesh=mesh,
        out_type=[
            jax.ShapeDtypeStruct((b * LPAD // q_chunk, q_chunk, HIDDEN),
                                 jnp.float32),
            jax.ShapeDtypeStruct((b * l // v_chunk, v_chunk, HIDDEN),
                                 jnp.float32),
        ],
        scratch_types=[
            pltpu.VMEM((q_nchunk, q_chunk), jnp.int32),
            pltpu.VMEM((v_nchunk, v_chunk), jnp.int32),
            [pltpu.VMEM((q_chunk, HIDDEN), jnp.float32)
             for _ in range(q_nbuf)],
            [pltpu.VMEM((v_chunk, HIDDEN), jnp.float32)
             for _ in range(v_nbuf)],
            pltpu.VMEM((l, HIDDEN), jnp.float32),
            [pltpu.SemaphoreType.DMA for _ in range(q_nbuf)],
            [pltpu.SemaphoreType.DMA for _ in range(q_nbuf)],
            [pltpu.SemaphoreType.DMA for _ in range(v_nbuf)],
            [pltpu.SemaphoreType.DMA for _ in range(v_nbuf)],
        ],
    )
    def k(qidx_hbm, vidx_hbm, tgt_hbm, src_hbm, pos_hbm, qout_hbm, vout_hbm,
          qidx_v, vidx_v, qbufs, vbufs, pos_v, qgs, qws, vgs, vws):
        wid = lax.axis_index("s") * 2 + lax.axis_index("c")
        pltpu.sync_copy(qidx_hbm.at[wid], qidx_v)
        pltpu.sync_copy(vidx_hbm.at[wid], vidx_v)
        pltpu.sync_copy(pos_hbm, pos_v)

        def q_gather(c, bf):
            pltpu.make_async_copy(tgt_hbm.at[qidx_v.at[c]], qbufs[bf],
                                  qgs[bf]).start()

        def v_gather(c, bf):
            pltpu.make_async_copy(src_hbm.at[vidx_v.at[c]], vbufs[bf],
                                  vgs[bf]).start()

        def q_emit(c, bf):
            pltpu.make_async_copy(tgt_hbm.at[qidx_v.at[c]], qbufs[bf],
                                  qgs[bf]).wait()
            wcp = pltpu.make_async_copy(qbufs[bf],
                                        qout_hbm.at[wid * q_nchunk + c],
                                        qws[bf])
            wcp.start()
            return wcp

        def v_emit(c, bf):
            g = wid * v_rows_per_w + c * v_chunk
            l_idx = g // b
            pltpu.make_async_copy(src_hbm.at[vidx_v.at[c]], vbufs[bf],
                                  vgs[bf]).wait()
            prows = [pos_v[l_idx, pl.ds(h * NLANE, NLANE)]
                     for h in range(HIDDEN // NLANE)]
            vbuf = vbufs[bf]

            def addrow(r, inner, vbuf=vbuf, prows=prows):
                for h in range(HIDDEN // NLANE):
                    sl = pl.ds(h * NLANE, NLANE)
                    vbuf[r, sl] = vbuf[r, sl] + prows[h]
                return inner

            lax.fori_loop(0, v_chunk, addrow, 0)
            wcp = pltpu.make_async_copy(vbuf,
                                        vout_hbm.at[wid * v_nchunk + c],
                                        vws[bf])
            wcp.start()
            return wcp

        for bf in range(q_nbuf):
            q_gather(bf, bf)
        for bf in range(v_nbuf):
            v_gather(bf, bf)

        def q_super(cc, carry):
            for bf in range(q_nbuf):
                c = cc * q_nbuf + bf
                wcp = q_emit(c, bf)
                wcp.wait()
                q_gather(c + q_nbuf, bf)
            return carry

        lax.fori_loop(0, (q_nchunk - q_nbuf) // q_nbuf, q_super, 0)
        for bf in range(q_nbuf):
            q_emit(q_nchunk - q_nbuf + bf, bf).wait()

        def v_super(cc, carry):
            for bf in range(v_nbuf):
                c = cc * v_nbuf + bf
                wcp = v_emit(c, bf)
                wcp.wait()
                v_gather(c + v_nbuf, bf)
            return carry

        lax.fori_loop(0, (v_nchunk - v_nbuf) // v_nbuf, v_super, 0)
        for bf in range(v_nbuf):
            v_emit(v_nchunk - v_nbuf + bf, bf).wait()

    return k


def _tc_matmul_call(b, l, nb):
    def body(ap_ref, qr_ref, pos_ref, out_ref):
        ap = ap_ref[...]
        qr = qr_ref[:, :l, :]
        acc = lax.dot_general(
            ap, qr, (((2,), (1,)), ((0,), (0,))),
            preferred_element_type=jnp.float32)
        out_ref[...] = jnp.transpose(acc, (1, 0, 2)) + pos_ref[...]

    return pl.pallas_call(
        body,
        grid=(b // nb,),
        in_specs=[
            pl.BlockSpec((nb, l, l), lambda i: (i, 0, 0)),
            pl.BlockSpec((nb, LPAD, HIDDEN), lambda i: (i, 0, 0)),
            pl.BlockSpec((l, 1, HIDDEN), lambda i: (0, 0, 0)),
        ],
        out_specs=pl.BlockSpec((l, nb, HIDDEN), lambda i: (0, i, 0)),
        out_shape=jax.ShapeDtypeStruct((l, b, HIDDEN), jnp.float32),
    )


def kernel(queries, values, absolute_positions, src_table, tgt_table):
    b, l = queries.shape
    nw = 32

    pad_cols = (jnp.arange(LPAD - l, dtype=jnp.int32)[None, :]
                + (LPAD - l) * jnp.arange(b, dtype=jnp.int32)[:, None])
    q_idx_pad = jnp.concatenate(
        [queries.astype(jnp.int32), pad_cols], axis=1)
    q_chunk = 2 * LPAD
    q_nchunk = b // (2 * nw)
    q_idx = q_idx_pad.reshape(nw, q_nchunk, q_chunk)

    v_chunk = 64
    v_nchunk = (b * l) // (nw * v_chunk)
    v_idx = values.astype(jnp.int32).T.reshape(nw, v_nchunk, v_chunk)

    pos = _pos_encoding(l, HIDDEN)

    sc = _sc_gather_call(b, l, nw, q_nchunk, q_chunk, v_nchunk, v_chunk)
    q_rows, v_emb_t = sc(q_idx, v_idx, tgt_table, src_table, pos)
    q_rows = q_rows.reshape(b * LPAD, HIDDEN)
    v_emb_t = v_emb_t.reshape(b * l, HIDDEN)

    tc = _tc_matmul_call(b, l, nb=64)
    q_emb_t = tc(absolute_positions, q_rows.reshape(b, LPAD, HIDDEN),
                 pos[:, None, :])

    q_emb = q_emb_t.transpose(1, 0, 2)
    v_emb = v_emb_t.reshape(l, b, HIDDEN).transpose(1, 0, 2)
    return q_emb, v_emb


# speedup vs baseline: 1.5906x; 1.0248x over previous
import functools

import jax
import jax.numpy as jnp
from jax import lax
from jax.experimental import pallas as pl
from jax.experimental.pallas import tpu as pltpu
from jax.experimental.pallas import tpu_sc as plsc

HIDDEN = 128
NLANE = 16
LPAD = 56


def _pos_encoding(length, hidden_size):
    pos = jnp.arange(length, dtype=jnp.float32)[:, None]
    i = jnp.arange(hidden_size // 2, dtype=jnp.float32)[None, :]
    angle_rates = 1.0 / jnp.power(10000.0, (2.0 * i) / jnp.float32(hidden_size))
    angles = pos * angle_rates
    return jnp.concatenate([jnp.sin(angles), jnp.cos(angles)], axis=-1)


def _sc_gather_call(b, l, nw, q_nchunk, q_chunk, v_nchunk, v_chunk):
    v_rows_per_w = b * l // nw
    mesh = plsc.VectorSubcoreMesh(core_axis_name="c", subcore_axis_name="s")

    q_nbuf = 2
    v_nbuf = 5
    assert (q_nchunk - q_nbuf) % q_nbuf == 0
    assert (v_nchunk - v_nbuf) % v_nbuf == 0

    @functools.partial(
        pl.kernel,
        mesh=mesh,
        out_type=[
            jax.ShapeDtypeStruct((b * LPAD // q_chunk, q_chunk, HIDDEN),
                                 jnp.float32),
            jax.ShapeDtypeStruct((b * l // v_chunk, v_chunk, HIDDEN),
                                 jnp.float32),
        ],
        scratch_types=[
            pltpu.VMEM((q_nchunk, q_chunk), jnp.int32),
            pltpu.VMEM((v_nchunk, v_chunk), jnp.int32),
            [pltpu.VMEM((q_chunk, HIDDEN), jnp.float32)
             for _ in range(q_nbuf)],
            [pltpu.VMEM((v_chunk, HIDDEN), jnp.float32)
             for _ in range(v_nbuf)],
            pltpu.VMEM((l, HIDDEN), jnp.float32),
            [pltpu.SemaphoreType.DMA for _ in range(q_nbuf)],
            [pltpu.SemaphoreType.DMA for _ in range(q_nbuf)],
            [pltpu.SemaphoreType.DMA for _ in range(v_nbuf)],
            [pltpu.SemaphoreType.DMA for _ in range(v_nbuf)],
        ],
    )
    def k(qidx_hbm, vidx_hbm, tgt_hbm, src_hbm, pos_hbm, qout_hbm, vout_hbm,
          qidx_v, vidx_v, qbufs, vbufs, pos_v, qgs, qws, vgs, vws):
        wid = lax.axis_index("s") * 2 + lax.axis_index("c")
        pltpu.sync_copy(qidx_hbm.at[wid], qidx_v)
        pltpu.sync_copy(vidx_hbm.at[wid], vidx_v)
        pltpu.sync_copy(pos_hbm, pos_v)

        def q_gather(c, bf):
            pltpu.make_async_copy(tgt_hbm.at[qidx_v.at[c]], qbufs[bf],
                                  qgs[bf]).start()

        def v_gather(c, bf):
            pltpu.make_async_copy(src_hbm.at[vidx_v.at[c]], vbufs[bf],
                                  vgs[bf]).start()

        def q_emit(c, bf):
            pltpu.make_async_copy(tgt_hbm.at[qidx_v.at[c]], qbufs[bf],
                                  qgs[bf]).wait()
            wcp = pltpu.make_async_copy(qbufs[bf],
                                        qout_hbm.at[wid * q_nchunk + c],
                                        qws[bf])
            wcp.start()
            return wcp

        def v_emit(c, bf):
            g = wid * v_rows_per_w + c * v_chunk
            l_idx = g // b
            pltpu.make_async_copy(src_hbm.at[vidx_v.at[c]], vbufs[bf],
                                  vgs[bf]).wait()
            prows = [pos_v[l_idx, pl.ds(h * NLANE, NLANE)]
                     for h in range(HIDDEN // NLANE)]
            vbuf = vbufs[bf]

            def addrow(r, inner, vbuf=vbuf, prows=prows):
                for h in range(HIDDEN // NLANE):
                    sl = pl.ds(h * NLANE, NLANE)
                    vbuf[r, sl] = vbuf[r, sl] + prows[h]
                return inner

            lax.fori_loop(0, v_chunk, addrow, 0)
            wcp = pltpu.make_async_copy(vbuf,
                                        vout_hbm.at[wid * v_nchunk + c],
                                        vws[bf])
            wcp.start()
            return wcp

        for bf in range(q_nbuf):
            q_gather(bf, bf)
        for bf in range(v_nbuf):
            v_gather(bf, bf)

        def q_super(cc, carry):
            for bf in range(q_nbuf):
                c = cc * q_nbuf + bf
                wcp = q_emit(c, bf)
                wcp.wait()
                q_gather(c + q_nbuf, bf)
            return carry

        lax.fori_loop(0, (q_nchunk - q_nbuf) // q_nbuf, q_super, 0)
        for bf in range(q_nbuf):
            q_emit(q_nchunk - q_nbuf + bf, bf).wait()

        def v_super(cc, carry):
            for bf in range(v_nbuf):
                c = cc * v_nbuf + bf
                wcp = v_emit(c, bf)
                wcp.wait()
                v_gather(c + v_nbuf, bf)
            return carry

        lax.fori_loop(0, (v_nchunk - v_nbuf) // v_nbuf, v_super, 0)
        for bf in range(v_nbuf):
            v_emit(v_nchunk - v_nbuf + bf, bf).wait()

    return k


def _tc_matmul_call(b, l, nb):
    def body(ap_ref, qr_ref, pos_ref, out_ref):
        ap = ap_ref[...]
        qr = qr_ref[:, :l, :]
        acc = lax.dot_general(
            ap, qr, (((2,), (1,)), ((0,), (0,))),
            preferred_element_type=jnp.float32)
        out_ref[...] = jnp.transpose(acc, (1, 0, 2)) + pos_ref[...]

    return pl.pallas_call(
        body,
        grid=(b // nb,),
        in_specs=[
            pl.BlockSpec((nb, l, l), lambda i: (i, 0, 0)),
            pl.BlockSpec((nb, LPAD, HIDDEN), lambda i: (i, 0, 0)),
            pl.BlockSpec((l, 1, HIDDEN), lambda i: (0, 0, 0)),
        ],
        out_specs=pl.BlockSpec((l, nb, HIDDEN), lambda i: (0, i, 0)),
        out_shape=jax.ShapeDtypeStruct((l, b, HIDDEN), jnp.float32),
    )


def kernel(queries, values, absolute_positions, src_table, tgt_table):
    b, l = queries.shape
    nw = 32

    pad_cols = (jnp.arange(LPAD - l, dtype=jnp.int32)[None, :]
                + (LPAD - l) * jnp.arange(b, dtype=jnp.int32)[:, None])
    q_idx_pad = jnp.concatenate(
        [queries.astype(jnp.int32), pad_cols], axis=1)
    q_chunk = 2 * LPAD
    q_nchunk = b // (2 * nw)
    q_idx = q_idx_pad.reshape(nw, q_nchunk, q_chunk)

    v_chunk = 64
    v_nchunk = (b * l) // (nw * v_chunk)
    v_idx = values.astype(jnp.int32).T.reshape(nw, v_nchunk, v_chunk)

    pos = _pos_encoding(l, HIDDEN)

    sc = _sc_gather_call(b, l, nw, q_nchunk, q_chunk, v_nchunk, v_chunk)
    q_rows, v_emb_t = sc(q_idx, v_idx, tgt_table, src_table, pos)
    q_rows = q_rows.reshape(b * LPAD, HIDDEN)
    v_emb_t = v_emb_t.reshape(b * l, HIDDEN)

    tc = _tc_matmul_call(b, l, nb=128)
    q_emb_t = tc(absolute_positions, q_rows.reshape(b, LPAD, HIDDEN),
                 pos[:, None, :])

    q_emb = q_emb_t.transpose(1, 0, 2)
    v_emb = v_emb_t.reshape(l, b, HIDDEN).transpose(1, 0, 2)
    return q_emb, v_emb


# speedup vs baseline: 1.6006x; 1.0063x over previous
import functools

import jax
import jax.numpy as jnp
from jax import lax
from jax.experimental import pallas as pl
from jax.experimental.pallas import tpu as pltpu
from jax.experimental.pallas import tpu_sc as plsc

HIDDEN = 128
NLANE = 16
LPAD = 56


def _pos_encoding(length, hidden_size):
    pos = jnp.arange(length, dtype=jnp.float32)[:, None]
    i = jnp.arange(hidden_size // 2, dtype=jnp.float32)[None, :]
    angle_rates = 1.0 / jnp.power(10000.0, (2.0 * i) / jnp.float32(hidden_size))
    angles = pos * angle_rates
    return jnp.concatenate([jnp.sin(angles), jnp.cos(angles)], axis=-1)


def _sc_gather_call(b, l, nw, q_nchunk, q_chunk, v_nchunk, v_chunk):
    v_rows_per_w = b * l // nw
    mesh = plsc.VectorSubcoreMesh(core_axis_name="c", subcore_axis_name="s")

    q_nbuf = 2
    v_nbuf = 5
    assert (q_nchunk - q_nbuf) % q_nbuf == 0
    assert (v_nchunk - v_nbuf) % v_nbuf == 0

    @functools.partial(
        pl.kernel,
        mesh=mesh,
        out_type=[
            jax.ShapeDtypeStruct((b * LPAD // q_chunk, q_chunk, HIDDEN),
                                 jnp.float32),
            jax.ShapeDtypeStruct((b * l // v_chunk, v_chunk, HIDDEN),
                                 jnp.float32),
        ],
        scratch_types=[
            pltpu.VMEM((q_nchunk, q_chunk), jnp.int32),
            pltpu.VMEM((v_nchunk, v_chunk), jnp.int32),
            [pltpu.VMEM((q_chunk, HIDDEN), jnp.float32)
             for _ in range(q_nbuf)],
            [pltpu.VMEM((v_chunk, HIDDEN), jnp.float32)
             for _ in range(v_nbuf)],
            pltpu.VMEM((l, HIDDEN), jnp.float32),
            [pltpu.SemaphoreType.DMA for _ in range(q_nbuf)],
            [pltpu.SemaphoreType.DMA for _ in range(q_nbuf)],
            [pltpu.SemaphoreType.DMA for _ in range(v_nbuf)],
            [pltpu.SemaphoreType.DMA for _ in range(v_nbuf)],
        ],
    )
    def k(qidx_hbm, vidx_hbm, tgt_hbm, src_hbm, pos_hbm, qout_hbm, vout_hbm,
          qidx_v, vidx_v, qbufs, vbufs, pos_v, qgs, qws, vgs, vws):
        wid = lax.axis_index("s") * 2 + lax.axis_index("c")
        pltpu.sync_copy(qidx_hbm.at[wid], qidx_v)
        pltpu.sync_copy(vidx_hbm.at[wid], vidx_v)
        pltpu.sync_copy(pos_hbm, pos_v)

        def q_gather(c, bf):
            pltpu.make_async_copy(tgt_hbm.at[qidx_v.at[c]], qbufs[bf],
                                  qgs[bf]).start()

        def v_gather(c, bf):
            pltpu.make_async_copy(src_hbm.at[vidx_v.at[c]], vbufs[bf],
                                  vgs[bf]).start()

        def q_emit(c, bf):
            pltpu.make_async_copy(tgt_hbm.at[qidx_v.at[c]], qbufs[bf],
                                  qgs[bf]).wait()
            wcp = pltpu.make_async_copy(qbufs[bf],
                                        qout_hbm.at[wid * q_nchunk + c],
                                        qws[bf])
            wcp.start()
            return wcp

        def v_emit(c, bf):
            g = wid * v_rows_per_w + c * v_chunk
            l_idx = g // b
            pltpu.make_async_copy(src_hbm.at[vidx_v.at[c]], vbufs[bf],
                                  vgs[bf]).wait()
            prows = [pos_v[l_idx, pl.ds(h * NLANE, NLANE)]
                     for h in range(HIDDEN // NLANE)]
            vbuf = vbufs[bf]

            def addrow(r, inner, vbuf=vbuf, prows=prows):
                for h in range(HIDDEN // NLANE):
                    sl = pl.ds(h * NLANE, NLANE)
                    vbuf[r, sl] = vbuf[r, sl] + prows[h]
                return inner

            lax.fori_loop(0, v_chunk, addrow, 0)
            wcp = pltpu.make_async_copy(vbuf,
                                        vout_hbm.at[wid * v_nchunk + c],
                                        vws[bf])
            wcp.start()
            return wcp

        for bf in range(q_nbuf):
            q_gather(bf, bf)
        for bf in range(v_nbuf):
            v_gather(bf, bf)

        def q_super(cc, carry):
            for bf in range(q_nbuf):
                c = cc * q_nbuf + bf
                wcp = q_emit(c, bf)
                wcp.wait()
                q_gather(c + q_nbuf, bf)
            return carry

        lax.fori_loop(0, (q_nchunk - q_nbuf) // q_nbuf, q_super, 0)
        for bf in range(q_nbuf):
            q_emit(q_nchunk - q_nbuf + bf, bf).wait()

        def v_super(cc, carry):
            for bf in range(v_nbuf):
                c = cc * v_nbuf + bf
                wcp = v_emit(c, bf)
                wcp.wait()
                v_gather(c + v_nbuf, bf)
            return carry

        lax.fori_loop(0, (v_nchunk - v_nbuf) // v_nbuf, v_super, 0)
        for bf in range(v_nbuf):
            v_emit(v_nchunk - v_nbuf + bf, bf).wait()

    return k


def _tc_matmul_call(b, l, nb):
    def body(ap_ref, qr_ref, pos_ref, out_ref):
        ap = ap_ref[...]
        qr = qr_ref[:, :l, :]
        acc = lax.dot_general(
            ap, qr, (((2,), (1,)), ((0,), (0,))),
            preferred_element_type=jnp.float32)
        out_ref[...] = jnp.transpose(acc, (1, 0, 2)) + pos_ref[...]

    return pl.pallas_call(
        body,
        grid=(b // nb,),
        in_specs=[
            pl.BlockSpec((nb, l, l), lambda i: (i, 0, 0)),
            pl.BlockSpec((nb, LPAD, HIDDEN), lambda i: (i, 0, 0)),
            pl.BlockSpec((l, 1, HIDDEN), lambda i: (0, 0, 0)),
        ],
        out_specs=pl.BlockSpec((l, nb, HIDDEN), lambda i: (0, i, 0)),
        out_shape=jax.ShapeDtypeStruct((l, b, HIDDEN), jnp.float32),
    )


def kernel(queries, values, absolute_positions, src_table, tgt_table):
    b, l = queries.shape
    nw = 32

    pad_cols = (jnp.arange(LPAD - l, dtype=jnp.int32)[None, :]
                + (LPAD - l) * jnp.arange(b, dtype=jnp.int32)[:, None])
    q_idx_pad = jnp.concatenate(
        [queries.astype(jnp.int32), pad_cols], axis=1)
    q_chunk = 2 * LPAD
    q_nchunk = b // (2 * nw)
    q_idx = q_idx_pad.reshape(nw, q_nchunk, q_chunk)

    v_chunk = 64
    v_nchunk = (b * l) // (nw * v_chunk)
    v_idx = values.astype(jnp.int32).T.reshape(nw, v_nchunk, v_chunk)

    pos = _pos_encoding(l, HIDDEN)

    sc = _sc_gather_call(b, l, nw, q_nchunk, q_chunk, v_nchunk, v_chunk)
    q_rows, v_emb_t = sc(q_idx, v_idx, tgt_table, src_table, pos)
    q_rows = q_rows.reshape(b * LPAD, HIDDEN)
    v_emb_t = v_emb_t.reshape(b * l, HIDDEN)

    tc = _tc_matmul_call(b, l, nb=256)
    q_emb_t = tc(absolute_positions, q_rows.reshape(b, LPAD, HIDDEN),
                 pos[:, None, :])

    q_emb = q_emb_t.transpose(1, 0, 2)
    v_emb = v_emb_t.reshape(l, b, HIDDEN).transpose(1, 0, 2)
    return q_emb, v_emb


# speedup vs baseline: 1.6502x; 1.0310x over previous
import functools

import jax
import jax.numpy as jnp
from jax import lax
from jax.experimental import pallas as pl
from jax.experimental.pallas import tpu as pltpu
from jax.experimental.pallas import tpu_sc as plsc

HIDDEN = 128
NLANE = 16
LPAD = 56


def _pos_encoding(length, hidden_size):
    pos = jnp.arange(length, dtype=jnp.float32)[:, None]
    i = jnp.arange(hidden_size // 2, dtype=jnp.float32)[None, :]
    angle_rates = 1.0 / jnp.power(10000.0, (2.0 * i) / jnp.float32(hidden_size))
    angles = pos * angle_rates
    return jnp.concatenate([jnp.sin(angles), jnp.cos(angles)], axis=-1)


def _sc_q_call(b, nw, q_nchunk, q_chunk):
    mesh = plsc.VectorSubcoreMesh(core_axis_name="c", subcore_axis_name="s")
    nbuf = 2
    assert q_nchunk % nbuf == 0

    @functools.partial(
        pl.kernel,
        mesh=mesh,
        out_type=jax.ShapeDtypeStruct((b * LPAD // q_chunk, q_chunk, HIDDEN),
                                      jnp.float32),
        scratch_types=[
            pltpu.VMEM((q_nchunk, q_chunk), jnp.int32),
            [pltpu.VMEM((q_chunk, HIDDEN), jnp.float32) for _ in range(nbuf)],
            [pltpu.SemaphoreType.DMA for _ in range(nbuf)],
            [pltpu.SemaphoreType.DMA for _ in range(nbuf)],
        ],
    )
    def k(qidx_hbm, tgt_hbm, qout_hbm, qidx_v, qbufs, qgs, qws):
        wid = lax.axis_index("s") * 2 + lax.axis_index("c")
        pltpu.sync_copy(qidx_hbm.at[wid], qidx_v)

        def gather(c, bf):
            pltpu.make_async_copy(tgt_hbm.at[qidx_v.at[c]], qbufs[bf],
                                  qgs[bf]).start()

        def emit(c, bf):
            pltpu.make_async_copy(tgt_hbm.at[qidx_v.at[c]], qbufs[bf],
                                  qgs[bf]).wait()
            wcp = pltpu.make_async_copy(qbufs[bf],
                                        qout_hbm.at[wid * q_nchunk + c],
                                        qws[bf])
            wcp.start()
            return wcp

        for bf in range(nbuf):
            gather(bf, bf)

        def super_it(cc, carry):
            for bf in range(nbuf):
                c = cc * nbuf + bf
                wcp = emit(c, bf)
                wcp.wait()
                gather(c + nbuf, bf)
            return carry

        lax.fori_loop(0, (q_nchunk - nbuf) // nbuf, super_it, 0)
        for bf in range(nbuf):
            emit(q_nchunk - nbuf + bf, bf).wait()

    return k


def _sc_v_call(b, l, nw, v_nchunk, v_chunk):
    v_rows_per_w = b * l // nw
    mesh = plsc.VectorSubcoreMesh(core_axis_name="c", subcore_axis_name="s")
    nbuf = 5
    assert v_nchunk % nbuf == 0

    @functools.partial(
        pl.kernel,
        mesh=mesh,
        out_type=jax.ShapeDtypeStruct((b * l // v_chunk, v_chunk, HIDDEN),
                                      jnp.float32),
        scratch_types=[
            pltpu.VMEM((v_nchunk, v_chunk), jnp.int32),
            [pltpu.VMEM((v_chunk, HIDDEN), jnp.float32) for _ in range(nbuf)],
            pltpu.VMEM((l, HIDDEN), jnp.float32),
            [pltpu.SemaphoreType.DMA for _ in range(nbuf)],
            [pltpu.SemaphoreType.DMA for _ in range(nbuf)],
        ],
    )
    def k(vidx_hbm, src_hbm, pos_hbm, vout_hbm, vidx_v, vbufs, pos_v, vgs,
          vws):
        wid = lax.axis_index("s") * 2 + lax.axis_index("c")
        pltpu.sync_copy(vidx_hbm.at[wid], vidx_v)
        pltpu.sync_copy(pos_hbm, pos_v)

        def gather(c, bf):
            pltpu.make_async_copy(src_hbm.at[vidx_v.at[c]], vbufs[bf],
                                  vgs[bf]).start()

        def emit(c, bf):
            g = wid * v_rows_per_w + c * v_chunk
            l_idx = g // b
            pltpu.make_async_copy(src_hbm.at[vidx_v.at[c]], vbufs[bf],
                                  vgs[bf]).wait()
            prows = [pos_v[l_idx, pl.ds(h * NLANE, NLANE)]
                     for h in range(HIDDEN // NLANE)]
            vbuf = vbufs[bf]

            def addrow(r, inner, vbuf=vbuf, prows=prows):
                for h in range(HIDDEN // NLANE):
                    sl = pl.ds(h * NLANE, NLANE)
                    vbuf[r, sl] = vbuf[r, sl] + prows[h]
                return inner

            lax.fori_loop(0, v_chunk, addrow, 0)
            wcp = pltpu.make_async_copy(vbuf,
                                        vout_hbm.at[wid * v_nchunk + c],
                                        vws[bf])
            wcp.start()
            return wcp

        for bf in range(nbuf):
            gather(bf, bf)

        def super_it(cc, carry):
            for bf in range(nbuf):
                c = cc * nbuf + bf
                wcp = emit(c, bf)
                wcp.wait()
                gather(c + nbuf, bf)
            return carry

        lax.fori_loop(0, (v_nchunk - nbuf) // nbuf, super_it, 0)
        for bf in range(nbuf):
            emit(v_nchunk - nbuf + bf, bf).wait()

    return k


def _tc_matmul_call(b, l, nb):
    def body(ap_ref, qr_ref, pos_ref, out_ref):
        ap = ap_ref[...]
        qr = qr_ref[:, :l, :]
        acc = lax.dot_general(
            ap, qr, (((2,), (1,)), ((0,), (0,))),
            preferred_element_type=jnp.float32)
        out_ref[...] = jnp.transpose(acc, (1, 0, 2)) + pos_ref[...]

    return pl.pallas_call(
        body,
        grid=(b // nb,),
        in_specs=[
            pl.BlockSpec((nb, l, l), lambda i: (i, 0, 0)),
            pl.BlockSpec((nb, LPAD, HIDDEN), lambda i: (i, 0, 0)),
            pl.BlockSpec((l, 1, HIDDEN), lambda i: (0, 0, 0)),
        ],
        out_specs=pl.BlockSpec((l, nb, HIDDEN), lambda i: (0, i, 0)),
        out_shape=jax.ShapeDtypeStruct((l, b, HIDDEN), jnp.float32),
    )


def kernel(queries, values, absolute_positions, src_table, tgt_table):
    b, l = queries.shape
    nw = 32

    pad_cols = (jnp.arange(LPAD - l, dtype=jnp.int32)[None, :]
                + (LPAD - l) * jnp.arange(b, dtype=jnp.int32)[:, None])
    q_idx_pad = jnp.concatenate(
        [queries.astype(jnp.int32), pad_cols], axis=1)
    q_chunk = 2 * LPAD
    q_nchunk = b // (2 * nw)
    q_idx = q_idx_pad.reshape(nw, q_nchunk, q_chunk)

    v_chunk = 64
    v_nchunk = (b * l) // (nw * v_chunk)
    v_idx = values.astype(jnp.int32).T.reshape(nw, v_nchunk, v_chunk)

    pos = _pos_encoding(l, HIDDEN)

    q_rows = _sc_q_call(b, nw, q_nchunk, q_chunk)(q_idx, tgt_table)
    v_emb_t = _sc_v_call(b, l, nw, v_nchunk, v_chunk)(v_idx, src_table, pos)
    q_rows = q_rows.reshape(b * LPAD, HIDDEN)
    v_emb_t = v_emb_t.reshape(b * l, HIDDEN)

    tc = _tc_matmul_call(b, l, nb=256)
    q_emb_t = tc(absolute_positions, q_rows.reshape(b, LPAD, HIDDEN),
                 pos[:, None, :])

    q_emb = q_emb_t.transpose(1, 0, 2)
    v_emb = v_emb_t.reshape(l, b, HIDDEN).transpose(1, 0, 2)
    return q_emb, v_emb
